# baseline (speedup 1.0000x reference)
import functools

import jax
import jax.numpy as jnp
from jax import lax
from jax.experimental import pallas as pl
from jax.experimental.pallas import tpu as pltpu
from jax.experimental.pallas import tpu_sc as plsc


def _router(x, Wg, *, N, D, E, K, CAP, ZROW, BT):
    NB = N // BT

    def body(x_ref, wg_ref, i0_ref, i1_ref, w0x_ref, w1x_ref, hist_ref):
        blk = pl.program_id(0)

        @pl.when(blk == 0)
        def _():
            hist_ref[...] = jnp.zeros_like(hist_ref)

        xb = x_ref[...]
        logits = jnp.dot(xb, wg_ref[...],
                         preferred_element_type=jnp.float32)
        cols = lax.broadcasted_iota(jnp.int32, (BT, E), 1)
        m0 = jnp.max(logits, axis=1, keepdims=True)
        idx0 = jnp.min(jnp.where(logits == m0, cols, E), axis=1,
                       keepdims=True)
        l2 = jnp.where(cols == idx0, -jnp.inf, logits)
        m1 = jnp.max(l2, axis=1, keepdims=True)
        idx1 = jnp.min(jnp.where(l2 == m1, cols, E), axis=1, keepdims=True)
        t = jnp.exp(m1 - m0)
        g0 = 1.0 / (1.0 + t)
        g1 = t / (1.0 + t)

        oh0 = (cols == idx0).astype(jnp.float32)
        oh1 = (cols == idx1).astype(jnp.float32)
        s = oh0 + oh1
        ti = lax.broadcasted_iota(jnp.int32, (BT, BT), 0)
        tj = lax.broadcasted_iota(jnp.int32, (BT, BT), 1)
        tri = (tj < ti).astype(jnp.float32)
        hist = hist_ref[0:1, 0:E]
        excl = jnp.dot(tri, s, preferred_element_type=jnp.float32) + hist
        pos0 = jnp.sum(excl * oh0, axis=1, keepdims=True)
        pos1 = jnp.sum(excl * oh1, axis=1, keepdims=True)
        hist_ref[0:1, 0:E] = hist + jnp.sum(s, axis=0, keepdims=True)

        keep0 = pos0 < CAP
        keep1 = pos1 < CAP
        p0 = pos0.astype(jnp.int32)
        p1 = pos1.astype(jnp.int32)
        i0 = jnp.where(keep0, idx0 * CAP + p0, ZROW)
        i1 = jnp.where(keep1, idx1 * CAP + p1, ZROW)
        w0 = jnp.where(keep0, g0, 0.0)
        w1 = jnp.where(keep1, g1, 0.0)

        i0_ref[...] = i0[:, 0]
        i1_ref[...] = i1[:, 0]
        w0x_ref[...] = jnp.broadcast_to(w0, (BT, 128))
        w1x_ref[...] = jnp.broadcast_to(w1, (BT, 128))

    return pl.pallas_call(
        body,
        grid=(NB,),
        in_specs=[
            pl.BlockSpec((BT, D), lambda i: (i, 0)),
            pl.BlockSpec((D, E), lambda i: (0, 0)),
        ],
        out_specs=[
            pl.BlockSpec((BT,), lambda i: (i,)),
            pl.BlockSpec((BT,), lambda i: (i,)),
            pl.BlockSpec((BT, 128), lambda i: (i, 0)),
            pl.BlockSpec((BT, 128), lambda i: (i, 0)),
        ],
        out_shape=[
            jax.ShapeDtypeStruct((N,), jnp.int32),
            jax.ShapeDtypeStruct((N,), jnp.int32),
            jax.ShapeDtypeStruct((N, 128), jnp.float32),
            jax.ShapeDtypeStruct((N, 128), jnp.float32),
        ],
        scratch_shapes=[pltpu.VMEM((8, 128), jnp.float32)],
    )(x, Wg)


def _dispatch(x, i0, i1, w0x, w1x, *, N, D, RD, NC, NW, TOKW):
    mesh = plsc.VectorSubcoreMesh(core_axis_name="c", subcore_axis_name="s")

    @functools.partial(
        pl.kernel,
        mesh=mesh,
        out_type=(
            jax.ShapeDtypeStruct((RD, D), jnp.float32),
            jax.ShapeDtypeStruct((RD, 128), jnp.float32),
        ),
        scratch_types=[
            pltpu.VMEM((TOKW, D), jnp.float32),
            pltpu.VMEM((TOKW, 128), jnp.float32),
            pltpu.VMEM((TOKW,), jnp.int32),
            pltpu.VMEM((TOKW,), jnp.int32),
            pltpu.SemaphoreType.DMA,
            pltpu.SemaphoreType.DMA,
        ],
    )
    def k(x_hbm, i0_hbm, i1_hbm, w0x_hbm, w1x_hbm, disp_hbm, wslot_hbm,
          xbuf, wbuf, i0v, i1v, sem, wsem):
        wid = lax.axis_index("s") * NC + lax.axis_index("c")
        base = wid * TOKW
        pltpu.sync_copy(i0_hbm.at[pl.ds(base, TOKW)], i0v)
        pltpu.sync_copy(i1_hbm.at[pl.ds(base, TOKW)], i1v)
        pltpu.sync_copy(x_hbm.at[pl.ds(base, TOKW)], xbuf)
        pltpu.sync_copy(w0x_hbm.at[pl.ds(base, TOKW)], wbuf)
        c0 = pltpu.async_copy(xbuf, disp_hbm.at[i0v], sem)
        c1 = pltpu.async_copy(wbuf, wslot_hbm.at[i0v], wsem)
        c2 = pltpu.async_copy(xbuf, disp_hbm.at[i1v], sem)
        c1.wait()
        pltpu.sync_copy(w1x_hbm.at[pl.ds(base, TOKW)], wbuf)
        c3 = pltpu.async_copy(wbuf, wslot_hbm.at[i1v], wsem)
        c0.wait()
        c2.wait()
        c3.wait()

    return k(x, i0, i1, w0x, w1x)


def _expert_mm(disp, We_b, be, wslot, *, D, E, CAP, RD):
    def body(disp_ref, we_ref, be_ref, ws_ref, y_ref):
        xb = disp_ref[...].astype(We_b.dtype)
        y = jnp.dot(xb, we_ref[0], preferred_element_type=jnp.float32)
        y_ref[...] = (y + be_ref[0]) * ws_ref[:, 0:1]

    return pl.pallas_call(
        body,
        grid=(E + 1,),
        in_specs=[
            pl.BlockSpec((CAP, D), lambda e: (e, 0)),
            pl.BlockSpec((1, D, D), lambda e: (jnp.minimum(e, E - 1), 0, 0)),
            pl.BlockSpec((1, 1, D), lambda e: (jnp.minimum(e, E - 1), 0, 0)),
            pl.BlockSpec((CAP, 128), lambda e: (e, 0)),
        ],
        out_specs=pl.BlockSpec((CAP, D), lambda e: (e, 0)),
        out_shape=jax.ShapeDtypeStruct((RD, D), jnp.float32),
    )(disp, We_b, be[:, None, :], wslot)


def _combine(y2d, i0, i1, *, N, D, RD, NC, NW, TOKW, CH):
    mesh = plsc.VectorSubcoreMesh(core_axis_name="c", subcore_axis_name="s")
    NCH = TOKW // CH
    NSEG = D // 16

    @functools.partial(
        pl.kernel,
        mesh=mesh,
        out_type=jax.ShapeDtypeStruct((N, D), jnp.float32),
        scratch_types=[
            pltpu.VMEM((CH, D), jnp.float32),
            pltpu.VMEM((CH, D), jnp.float32),
            pltpu.VMEM((NCH, CH), jnp.int32),
            pltpu.VMEM((NCH, CH), jnp.int32),
            pltpu.SemaphoreType.DMA,
            pltpu.SemaphoreType.DMA,
        ],
    )
    def k(y_hbm, i0_hbm, i1_hbm, out_hbm, buf0, buf1, i0v, i1v, s0, s1):
        wid = lax.axis_index("s") * NC + lax.axis_index("c")
        base = wid * TOKW
        pltpu.sync_copy(i0_hbm.at[wid], i0v)
        pltpu.sync_copy(i1_hbm.at[wid], i1v)
        for c in range(NCH):
            g0 = pltpu.async_copy(y_hbm.at[i0v.at[c]], buf0, s0)
            g1 = pltpu.async_copy(y_hbm.at[i1v.at[c]], buf1, s1)
            g0.wait()
            g1.wait()

            def row_body(j, _):
                def seg_body(s, _):
                    sl = pl.ds(s * 16, 16)
                    buf0[j, sl] = buf0[j, sl] + buf1[j, sl]
                    return 0
                return lax.fori_loop(0, NSEG, seg_body, 0)

            lax.fori_loop(0, CH, row_body, 0)
            pltpu.sync_copy(buf0, out_hbm.at[pl.ds(base + c * CH, CH)])

    return k(y2d, i0.reshape(NW, NCH, CH), i1.reshape(NW, NCH, CH))


def kernel(hidden_states, Wg, We, be):
    x = hidden_states
    N, D = x.shape
    E = Wg.shape[1]
    K = 2
    CAP = int(K * N / E * 1.25)
    ZROW = E * CAP
    RD = (E + 1) * CAP
    BT = 512
    info = plsc.get_sparse_core_info()
    NC, NS = info.num_cores, info.num_subcores
    NW = NC * NS
    TOKW = N // NW

    i0, i1, w0x, w1x = _router(x, Wg, N=N, D=D, E=E, K=K, CAP=CAP,
                               ZROW=ZROW, BT=BT)
    disp, wslot = _dispatch(x, i0, i1, w0x, w1x, N=N, D=D, RD=RD,
                            NC=NC, NW=NW, TOKW=TOKW)
    y2d = _expert_mm(disp, We.astype(jnp.bfloat16), be, wslot,
                     D=D, E=E, CAP=CAP, RD=RD)
    out = _combine(y2d, i0, i1, N=N, D=D, RD=RD, NC=NC, NW=NW, TOKW=TOKW,
                   CH=64)
    return out


# speedup vs baseline: 1.3063x; 1.3063x over previous
import functools

import jax
import jax.numpy as jnp
from jax import lax
from jax.experimental import pallas as pl
from jax.experimental.pallas import tpu as pltpu
from jax.experimental.pallas import tpu_sc as plsc


def _router(x, Wg, *, N, D, E, K, CAP, ZROW, BT):
    NB = N // BT

    def body(x_ref, wg_ref, i0_ref, i1_ref, w0x_ref, w1x_ref, hist_ref):
        blk = pl.program_id(0)

        @pl.when(blk == 0)
        def _():
            hist_ref[...] = jnp.zeros_like(hist_ref)

        xb = x_ref[...]
        logits = jnp.dot(xb, wg_ref[...],
                         preferred_element_type=jnp.float32)
        cols = lax.broadcasted_iota(jnp.int32, (BT, E), 1)
        m0 = jnp.max(logits, axis=1, keepdims=True)
        idx0 = jnp.min(jnp.where(logits == m0, cols, E), axis=1,
                       keepdims=True)
        l2 = jnp.where(cols == idx0, -jnp.inf, logits)
        m1 = jnp.max(l2, axis=1, keepdims=True)
        idx1 = jnp.min(jnp.where(l2 == m1, cols, E), axis=1, keepdims=True)
        t = jnp.exp(m1 - m0)
        g0 = 1.0 / (1.0 + t)
        g1 = t / (1.0 + t)

        oh0 = (cols == idx0).astype(jnp.float32)
        oh1 = (cols == idx1).astype(jnp.float32)
        s = oh0 + oh1
        ti = lax.broadcasted_iota(jnp.int32, (BT, BT), 0)
        tj = lax.broadcasted_iota(jnp.int32, (BT, BT), 1)
        tri = (tj < ti).astype(jnp.float32)
        hist = hist_ref[0:1, 0:E]
        excl = jnp.dot(tri, s, preferred_element_type=jnp.float32) + hist
        pos0 = jnp.sum(excl * oh0, axis=1, keepdims=True)
        pos1 = jnp.sum(excl * oh1, axis=1, keepdims=True)
        hist_ref[0:1, 0:E] = hist + jnp.sum(s, axis=0, keepdims=True)

        keep0 = pos0 < CAP
        keep1 = pos1 < CAP
        p0 = pos0.astype(jnp.int32)
        p1 = pos1.astype(jnp.int32)
        i0 = jnp.where(keep0, idx0 * CAP + p0, ZROW)
        i1 = jnp.where(keep1, idx1 * CAP + p1, ZROW)
        w0 = jnp.where(keep0, g0, 0.0)
        w1 = jnp.where(keep1, g1, 0.0)

        i0_ref[...] = i0[:, 0]
        i1_ref[...] = i1[:, 0]
        w0x_ref[...] = jnp.broadcast_to(w0, (BT, 128))
        w1x_ref[...] = jnp.broadcast_to(w1, (BT, 128))

    return pl.pallas_call(
        body,
        grid=(NB,),
        in_specs=[
            pl.BlockSpec((BT, D), lambda i: (i, 0)),
            pl.BlockSpec((D, E), lambda i: (0, 0)),
        ],
        out_specs=[
            pl.BlockSpec((BT,), lambda i: (i,)),
            pl.BlockSpec((BT,), lambda i: (i,)),
            pl.BlockSpec((BT, 128), lambda i: (i, 0)),
            pl.BlockSpec((BT, 128), lambda i: (i, 0)),
        ],
        out_shape=[
            jax.ShapeDtypeStruct((N,), jnp.int32),
            jax.ShapeDtypeStruct((N,), jnp.int32),
            jax.ShapeDtypeStruct((N, 128), jnp.float32),
            jax.ShapeDtypeStruct((N, 128), jnp.float32),
        ],
        scratch_shapes=[pltpu.VMEM((8, 128), jnp.float32)],
    )(x, Wg)


def _dispatch(x, i0, i1, w0x, w1x, *, N, D, RD, NC, NW, TOKW):
    mesh = plsc.VectorSubcoreMesh(core_axis_name="c", subcore_axis_name="s")

    @functools.partial(
        pl.kernel,
        mesh=mesh,
        out_type=(
            jax.ShapeDtypeStruct((RD, D), jnp.float32),
            jax.ShapeDtypeStruct((RD, 128), jnp.float32),
        ),
        scratch_types=[
            pltpu.VMEM((TOKW, D), jnp.float32),
            pltpu.VMEM((TOKW, 128), jnp.float32),
            pltpu.VMEM((TOKW,), jnp.int32),
            pltpu.VMEM((TOKW,), jnp.int32),
            pltpu.SemaphoreType.DMA,
            pltpu.SemaphoreType.DMA,
        ],
    )
    def k(x_hbm, i0_hbm, i1_hbm, w0x_hbm, w1x_hbm, disp_hbm, wslot_hbm,
          xbuf, wbuf, i0v, i1v, sem, wsem):
        wid = lax.axis_index("s") * NC + lax.axis_index("c")
        base = wid * TOKW
        pltpu.sync_copy(i0_hbm.at[pl.ds(base, TOKW)], i0v)
        pltpu.sync_copy(i1_hbm.at[pl.ds(base, TOKW)], i1v)
        pltpu.sync_copy(x_hbm.at[pl.ds(base, TOKW)], xbuf)
        pltpu.sync_copy(w0x_hbm.at[pl.ds(base, TOKW)], wbuf)
        c0 = pltpu.async_copy(xbuf, disp_hbm.at[i0v], sem)
        c1 = pltpu.async_copy(wbuf, wslot_hbm.at[i0v], wsem)
        c2 = pltpu.async_copy(xbuf, disp_hbm.at[i1v], sem)
        c1.wait()
        pltpu.sync_copy(w1x_hbm.at[pl.ds(base, TOKW)], wbuf)
        c3 = pltpu.async_copy(wbuf, wslot_hbm.at[i1v], wsem)
        c0.wait()
        c2.wait()
        c3.wait()

    return k(x, i0, i1, w0x, w1x)


def _expert_mm(disp, We_b, be, wslot, *, D, E, CAP, RD):
    def body(disp_ref, we_ref, be_ref, ws_ref, y_ref):
        xb = disp_ref[...].astype(We_b.dtype)
        y = jnp.dot(xb, we_ref[0], preferred_element_type=jnp.float32)
        y_ref[...] = (y + be_ref[0]) * ws_ref[:, 0:1]

    return pl.pallas_call(
        body,
        grid=(E + 1,),
        in_specs=[
            pl.BlockSpec((CAP, D), lambda e: (e, 0)),
            pl.BlockSpec((1, D, D), lambda e: (jnp.minimum(e, E - 1), 0, 0)),
            pl.BlockSpec((1, 1, D), lambda e: (jnp.minimum(e, E - 1), 0, 0)),
            pl.BlockSpec((CAP, 128), lambda e: (e, 0)),
        ],
        out_specs=pl.BlockSpec((CAP, D), lambda e: (e, 0)),
        out_shape=jax.ShapeDtypeStruct((RD, D), jnp.float32),
    )(disp, We_b, be[:, None, :], wslot)


def _combine(y2d, i0, i1, *, N, D, RD, NC, NW, TOKW, CH):
    mesh = plsc.VectorSubcoreMesh(core_axis_name="c", subcore_axis_name="s")
    NCH = TOKW // CH
    NSEG = D // 16

    @functools.partial(
        pl.kernel,
        mesh=mesh,
        out_type=jax.ShapeDtypeStruct((N, D), jnp.float32),
        scratch_types=[
            pltpu.VMEM((CH, D), jnp.float32),
            pltpu.VMEM((CH, D), jnp.float32),
            pltpu.VMEM((NCH, CH), jnp.int32),
            pltpu.VMEM((NCH, CH), jnp.int32),
            pltpu.SemaphoreType.DMA,
            pltpu.SemaphoreType.DMA,
        ],
    )
    def k(y_hbm, i0_hbm, i1_hbm, out_hbm, buf0, buf1, i0v, i1v, s0, s1):
        wid = lax.axis_index("s") * NC + lax.axis_index("c")
        base = wid * TOKW
        pltpu.sync_copy(i0_hbm.at[wid], i0v)
        pltpu.sync_copy(i1_hbm.at[wid], i1v)
        for c in range(NCH):
            g0 = pltpu.async_copy(y_hbm.at[i0v.at[c]], buf0, s0)
            g1 = pltpu.async_copy(y_hbm.at[i1v.at[c]], buf1, s1)
            g0.wait()
            g1.wait()

            def row_body(j, _):
                def seg_body(s, _):
                    sl = pl.ds(s * 16, 16)
                    buf0[j, sl] = buf0[j, sl] + buf1[j, sl]
                    return 0
                return lax.fori_loop(0, NSEG, seg_body, 0)

            lax.fori_loop(0, CH, row_body, 0)
            pltpu.sync_copy(buf0, out_hbm.at[pl.ds(base + c * CH, CH)])

    return k(y2d, i0.reshape(NW, NCH, CH), i1.reshape(NW, NCH, CH))


def kernel(hidden_states, Wg, We, be):
    x = hidden_states
    N, D = x.shape
    E = Wg.shape[1]
    K = 2
    CAP = int(K * N / E * 1.25)
    ZROW = E * CAP
    RD = (E + 1) * CAP
    BT = 512
    info = plsc.get_sparse_core_info()
    NC, NS = info.num_cores, info.num_subcores
    NW = NC * NS
    TOKW = N // NW

    i0, i1, w0x, w1x = _router(x, Wg, N=N, D=D, E=E, K=K, CAP=CAP,
                               ZROW=ZROW, BT=BT)
    disp, wslot = _dispatch(x, i0, i1, w0x, w1x, N=N, D=D, RD=RD,
                            NC=NC, NW=NW, TOKW=TOKW)
    y2d = _expert_mm(disp, We, be, wslot, D=D, E=E, CAP=CAP, RD=RD)
    out = _combine(y2d, i0, i1, N=N, D=D, RD=RD, NC=NC, NW=NW, TOKW=TOKW,
                   CH=64)
    return out


# speedup vs baseline: 1.4766x; 1.1304x over previous
import functools

import jax
import jax.numpy as jnp
from jax import lax
from jax.experimental import pallas as pl
from jax.experimental.pallas import tpu as pltpu
from jax.experimental.pallas import tpu_sc as plsc


def _router(x, Wg, *, N, D, E, K, CAP, ZROW, BT):
    NB = N // BT

    def body(x_ref, wg_ref, i0_ref, i1_ref, w0x_ref, w1x_ref, hist_ref):
        blk = pl.program_id(0)

        @pl.when(blk == 0)
        def _():
            hist_ref[...] = jnp.zeros_like(hist_ref)

        xb = x_ref[...]
        logits = jnp.dot(xb, wg_ref[...],
                         preferred_element_type=jnp.float32)
        cols = lax.broadcasted_iota(jnp.int32, (BT, E), 1)
        m0 = jnp.max(logits, axis=1, keepdims=True)
        idx0 = jnp.min(jnp.where(logits == m0, cols, E), axis=1,
                       keepdims=True)
        l2 = jnp.where(cols == idx0, -jnp.inf, logits)
        m1 = jnp.max(l2, axis=1, keepdims=True)
        idx1 = jnp.min(jnp.where(l2 == m1, cols, E), axis=1, keepdims=True)
        t = jnp.exp(m1 - m0)
        g0 = 1.0 / (1.0 + t)
        g1 = t / (1.0 + t)

        oh0 = (cols == idx0).astype(jnp.float32)
        oh1 = (cols == idx1).astype(jnp.float32)
        s = oh0 + oh1
        ti = lax.broadcasted_iota(jnp.int32, (BT, BT), 0)
        tj = lax.broadcasted_iota(jnp.int32, (BT, BT), 1)
        tri = (tj < ti).astype(jnp.float32)
        hist = hist_ref[0:1, 0:E]
        excl = jnp.dot(tri, s, preferred_element_type=jnp.float32) + hist
        pos0 = jnp.sum(excl * oh0, axis=1, keepdims=True)
        pos1 = jnp.sum(excl * oh1, axis=1, keepdims=True)
        hist_ref[0:1, 0:E] = hist + jnp.sum(s, axis=0, keepdims=True)

        keep0 = pos0 < CAP
        keep1 = pos1 < CAP
        p0 = pos0.astype(jnp.int32)
        p1 = pos1.astype(jnp.int32)
        i0 = jnp.where(keep0, idx0 * CAP + p0, ZROW)
        i1 = jnp.where(keep1, idx1 * CAP + p1, ZROW)
        w0 = jnp.where(keep0, g0, 0.0)
        w1 = jnp.where(keep1, g1, 0.0)

        i0_ref[...] = i0[:, 0]
        i1_ref[...] = i1[:, 0]
        w0x_ref[...] = jnp.broadcast_to(w0, (BT, 128))
        w1x_ref[...] = jnp.broadcast_to(w1, (BT, 128))

    return pl.pallas_call(
        body,
        grid=(NB,),
        in_specs=[
            pl.BlockSpec((BT, D), lambda i: (i, 0)),
            pl.BlockSpec((D, E), lambda i: (0, 0)),
        ],
        out_specs=[
            pl.BlockSpec((BT,), lambda i: (i,)),
            pl.BlockSpec((BT,), lambda i: (i,)),
            pl.BlockSpec((BT, 128), lambda i: (i, 0)),
            pl.BlockSpec((BT, 128), lambda i: (i, 0)),
        ],
        out_shape=[
            jax.ShapeDtypeStruct((N,), jnp.int32),
            jax.ShapeDtypeStruct((N,), jnp.int32),
            jax.ShapeDtypeStruct((N, 128), jnp.float32),
            jax.ShapeDtypeStruct((N, 128), jnp.float32),
        ],
        scratch_shapes=[pltpu.VMEM((8, 128), jnp.float32)],
    )(x, Wg)


def _dispatch(x, i0, i1, w0x, w1x, *, N, D, RD, NC, NW, TOKW):
    mesh = plsc.VectorSubcoreMesh(core_axis_name="c", subcore_axis_name="s")

    @functools.partial(
        pl.kernel,
        mesh=mesh,
        out_type=(
            jax.ShapeDtypeStruct((RD, D), jnp.float32),
            jax.ShapeDtypeStruct((RD, 128), jnp.float32),
        ),
        scratch_types=[
            pltpu.VMEM((TOKW, D), jnp.float32),
            pltpu.VMEM((TOKW, 128), jnp.float32),
            pltpu.VMEM((TOKW,), jnp.int32),
            pltpu.VMEM((TOKW,), jnp.int32),
            pltpu.SemaphoreType.DMA,
            pltpu.SemaphoreType.DMA,
        ],
    )
    def k(x_hbm, i0_hbm, i1_hbm, w0x_hbm, w1x_hbm, disp_hbm, wslot_hbm,
          xbuf, wbuf, i0v, i1v, sem, wsem):
        wid = lax.axis_index("s") * NC + lax.axis_index("c")
        base = wid * TOKW
        pltpu.sync_copy(i0_hbm.at[pl.ds(base, TOKW)], i0v)
        pltpu.sync_copy(i1_hbm.at[pl.ds(base, TOKW)], i1v)
        pltpu.sync_copy(x_hbm.at[pl.ds(base, TOKW)], xbuf)
        pltpu.sync_copy(w0x_hbm.at[pl.ds(base, TOKW)], wbuf)
        c0 = pltpu.async_copy(xbuf, disp_hbm.at[i0v], sem)
        c1 = pltpu.async_copy(wbuf, wslot_hbm.at[i0v], wsem)
        c2 = pltpu.async_copy(xbuf, disp_hbm.at[i1v], sem)
        c1.wait()
        pltpu.sync_copy(w1x_hbm.at[pl.ds(base, TOKW)], wbuf)
        c3 = pltpu.async_copy(wbuf, wslot_hbm.at[i1v], wsem)
        c0.wait()
        c2.wait()
        c3.wait()

    return k(x, i0, i1, w0x, w1x)


def _expert_mm(disp, We_b, be, wslot, *, D, E, CAP, RD):
    def body(disp_ref, we_ref, be_ref, ws_ref, y_ref):
        xb = disp_ref[...].astype(We_b.dtype)
        y = jnp.dot(xb, we_ref[0], preferred_element_type=jnp.float32)
        y_ref[...] = (y + be_ref[0]) * ws_ref[:, 0:1]

    return pl.pallas_call(
        body,
        grid=(E + 1,),
        in_specs=[
            pl.BlockSpec((CAP, D), lambda e: (e, 0)),
            pl.BlockSpec((1, D, D), lambda e: (jnp.minimum(e, E - 1), 0, 0)),
            pl.BlockSpec((1, 1, D), lambda e: (jnp.minimum(e, E - 1), 0, 0)),
            pl.BlockSpec((CAP, 128), lambda e: (e, 0)),
        ],
        out_specs=pl.BlockSpec((CAP, D), lambda e: (e, 0)),
        out_shape=jax.ShapeDtypeStruct((RD, D), jnp.float32),
    )(disp, We_b, be[:, None, :], wslot)


def _combine(y2d, i0, i1, *, N, D, RD, NC, NW, TOKW, CH):
    mesh = plsc.VectorSubcoreMesh(core_axis_name="c", subcore_axis_name="s")
    NCH = TOKW // CH
    NSEG = D // 16

    @functools.partial(
        pl.kernel,
        mesh=mesh,
        out_type=jax.ShapeDtypeStruct((N, D), jnp.float32),
        scratch_types=[
            pltpu.VMEM((2, CH, D), jnp.float32),
            pltpu.VMEM((2, CH, D), jnp.float32),
            pltpu.VMEM((2, CH, D), jnp.float32),
            pltpu.VMEM((NCH, CH), jnp.int32),
            pltpu.VMEM((NCH, CH), jnp.int32),
            pltpu.SemaphoreType.DMA,
            pltpu.SemaphoreType.DMA,
            pltpu.SemaphoreType.DMA,
            pltpu.SemaphoreType.DMA,
            pltpu.SemaphoreType.DMA,
            pltpu.SemaphoreType.DMA,
        ],
    )
    def k(y_hbm, i0_hbm, i1_hbm, out_hbm, buf0, buf1, obuf, i0v, i1v,
          sg0a, sg1a, sg0b, sg1b, swa, swb):
        wid = lax.axis_index("s") * NC + lax.axis_index("c")
        base = wid * TOKW
        pltpu.sync_copy(i0_hbm.at[wid], i0v)
        pltpu.sync_copy(i1_hbm.at[wid], i1v)
        sg = ((sg0a, sg1a), (sg0b, sg1b))
        sw = (swa, swb)

        def issue(c):
            p = c % 2
            g0 = pltpu.async_copy(y_hbm.at[i0v.at[c]], buf0.at[p], sg[p][0])
            g1 = pltpu.async_copy(y_hbm.at[i1v.at[c]], buf1.at[p], sg[p][1])
            return g0, g1

        descs = {0: issue(0)}
        if NCH > 1:
            descs[1] = issue(1)
        writes = {}
        for c in range(NCH):
            p = c % 2
            g0, g1 = descs[c]
            g0.wait()
            g1.wait()
            if c >= 2:
                writes[c - 2].wait()

            def row_body(j, _, p=p):
                for s in range(NSEG):
                    sl = pl.ds(s * 16, 16)
                    obuf[p, j, sl] = buf0[p, j, sl] + buf1[p, j, sl]
                return 0

            lax.fori_loop(0, CH, row_body, 0)
            writes[c] = pltpu.async_copy(
                obuf.at[p], out_hbm.at[pl.ds(base + c * CH, CH)], sw[p])
            if c + 2 < NCH:
                descs[c + 2] = issue(c + 2)
        if NCH > 1:
            writes[NCH - 2].wait()
        writes[NCH - 1].wait()

    return k(y2d, i0.reshape(NW, NCH, CH), i1.reshape(NW, NCH, CH))


def kernel(hidden_states, Wg, We, be):
    x = hidden_states
    N, D = x.shape
    E = Wg.shape[1]
    K = 2
    CAP = int(K * N / E * 1.25)
    ZROW = E * CAP
    RD = (E + 1) * CAP
    BT = 512
    info = plsc.get_sparse_core_info()
    NC, NS = info.num_cores, info.num_subcores
    NW = NC * NS
    TOKW = N // NW

    i0, i1, w0x, w1x = _router(x, Wg, N=N, D=D, E=E, K=K, CAP=CAP,
                               ZROW=ZROW, BT=BT)
    disp, wslot = _dispatch(x, i0, i1, w0x, w1x, N=N, D=D, RD=RD,
                            NC=NC, NW=NW, TOKW=TOKW)
    y2d = _expert_mm(disp, We, be, wslot, D=D, E=E, CAP=CAP, RD=RD)
    out = _combine(y2d, i0, i1, N=N, D=D, RD=RD, NC=NC, NW=NW, TOKW=TOKW,
                   CH=16)
    return out


# speedup vs baseline: 1.6547x; 1.1206x over previous
import functools

import jax
import jax.numpy as jnp
from jax import lax
from jax.experimental import pallas as pl
from jax.experimental.pallas import tpu as pltpu
from jax.experimental.pallas import tpu_sc as plsc


def _router(x, Wg, *, N, D, E, K, CAP, ZROW, BT):
    NB = N // BT

    def body(x_ref, wg_ref, i0_ref, i1_ref, w0x_ref, w1x_ref, hist_ref):
        blk = pl.program_id(0)

        @pl.when(blk == 0)
        def _():
            hist_ref[...] = jnp.zeros_like(hist_ref)

        xb = x_ref[...]
        logits = jnp.dot(xb, wg_ref[...],
                         preferred_element_type=jnp.float32)
        cols = lax.broadcasted_iota(jnp.int32, (BT, E), 1)
        m0 = jnp.max(logits, axis=1, keepdims=True)
        idx0 = jnp.min(jnp.where(logits == m0, cols, E), axis=1,
                       keepdims=True)
        l2 = jnp.where(cols == idx0, -jnp.inf, logits)
        m1 = jnp.max(l2, axis=1, keepdims=True)
        idx1 = jnp.min(jnp.where(l2 == m1, cols, E), axis=1, keepdims=True)
        t = jnp.exp(m1 - m0)
        g0 = 1.0 / (1.0 + t)
        g1 = t / (1.0 + t)

        oh0 = (cols == idx0).astype(jnp.float32)
        oh1 = (cols == idx1).astype(jnp.float32)
        s = oh0 + oh1
        ti = lax.broadcasted_iota(jnp.int32, (BT, BT), 0)
        tj = lax.broadcasted_iota(jnp.int32, (BT, BT), 1)
        tri = (tj < ti).astype(jnp.float32)
        hist = hist_ref[0:1, 0:E]
        excl = jnp.dot(tri, s, preferred_element_type=jnp.float32) + hist
        pos0 = jnp.sum(excl * oh0, axis=1, keepdims=True)
        pos1 = jnp.sum(excl * oh1, axis=1, keepdims=True)
        hist_ref[0:1, 0:E] = hist + jnp.sum(s, axis=0, keepdims=True)

        keep0 = pos0 < CAP
        keep1 = pos1 < CAP
        p0 = pos0.astype(jnp.int32)
        p1 = pos1.astype(jnp.int32)
        i0 = jnp.where(keep0, idx0 * CAP + p0, ZROW)
        i1 = jnp.where(keep1, idx1 * CAP + p1, ZROW)
        w0 = jnp.where(keep0, g0, 0.0)
        w1 = jnp.where(keep1, g1, 0.0)

        i0_ref[...] = i0[:, 0]
        i1_ref[...] = i1[:, 0]
        w0x_ref[...] = jnp.broadcast_to(w0, (BT, 128))
        w1x_ref[...] = jnp.broadcast_to(w1, (BT, 128))

    return pl.pallas_call(
        body,
        grid=(NB,),
        in_specs=[
            pl.BlockSpec((BT, D), lambda i: (i, 0)),
            pl.BlockSpec((D, E), lambda i: (0, 0)),
        ],
        out_specs=[
            pl.BlockSpec((BT,), lambda i: (i,)),
            pl.BlockSpec((BT,), lambda i: (i,)),
            pl.BlockSpec((BT, 128), lambda i: (i, 0)),
            pl.BlockSpec((BT, 128), lambda i: (i, 0)),
        ],
        out_shape=[
            jax.ShapeDtypeStruct((N,), jnp.int32),
            jax.ShapeDtypeStruct((N,), jnp.int32),
            jax.ShapeDtypeStruct((N, 128), jnp.float32),
            jax.ShapeDtypeStruct((N, 128), jnp.float32),
        ],
        scratch_shapes=[pltpu.VMEM((8, 128), jnp.float32)],
    )(x, Wg)


def _dispatch(x, i0, i1, w0x, w1x, *, N, D, RD, NC, NW, TOKW):
    mesh = plsc.VectorSubcoreMesh(core_axis_name="c", subcore_axis_name="s")

    @functools.partial(
        pl.kernel,
        mesh=mesh,
        out_type=(
            jax.ShapeDtypeStruct((RD, D), jnp.float32),
            jax.ShapeDtypeStruct((RD, 128), jnp.float32),
        ),
        scratch_types=[
            pltpu.VMEM((TOKW, D), jnp.float32),
            pltpu.VMEM((TOKW, 128), jnp.float32),
            pltpu.VMEM((TOKW,), jnp.int32),
            pltpu.VMEM((TOKW,), jnp.int32),
            pltpu.SemaphoreType.DMA,
            pltpu.SemaphoreType.DMA,
        ],
    )
    def k(x_hbm, i0_hbm, i1_hbm, w0x_hbm, w1x_hbm, disp_hbm, wslot_hbm,
          xbuf, wbuf, i0v, i1v, sem, wsem):
        wid = lax.axis_index("s") * NC + lax.axis_index("c")
        base = wid * TOKW
        pltpu.sync_copy(i0_hbm.at[pl.ds(base, TOKW)], i0v)
        pltpu.sync_copy(i1_hbm.at[pl.ds(base, TOKW)], i1v)
        pltpu.sync_copy(x_hbm.at[pl.ds(base, TOKW)], xbuf)
        pltpu.sync_copy(w0x_hbm.at[pl.ds(base, TOKW)], wbuf)
        c0 = pltpu.async_copy(xbuf, disp_hbm.at[i0v], sem)
        c1 = pltpu.async_copy(wbuf, wslot_hbm.at[i0v], wsem)
        c2 = pltpu.async_copy(xbuf, disp_hbm.at[i1v], sem)
        c1.wait()
        pltpu.sync_copy(w1x_hbm.at[pl.ds(base, TOKW)], wbuf)
        c3 = pltpu.async_copy(wbuf, wslot_hbm.at[i1v], wsem)
        c0.wait()
        c2.wait()
        c3.wait()

    return k(x, i0, i1, w0x, w1x)


def _expert_mm(disp, We_b, be, wslot, *, D, E, CAP, RD, P):
    G = (E + P) // P

    def body(disp_ref, we_ref, be_ref, ws_ref, y_ref):
        for p in range(P):
            rows = pl.ds(p * CAP, CAP)
            xb = disp_ref[rows, :].astype(We_b.dtype)
            y = jnp.dot(xb, we_ref[p], preferred_element_type=jnp.float32)
            y_ref[rows, :] = (y + be_ref[p]) * ws_ref[rows, 0:1]

    return pl.pallas_call(
        body,
        grid=(G,),
        in_specs=[
            pl.BlockSpec((P * CAP, D), lambda e: (e, 0)),
            pl.BlockSpec((P, D, D), lambda e: (jnp.minimum(e, E // P - 1), 0, 0)),
            pl.BlockSpec((P, 1, D), lambda e: (jnp.minimum(e, E // P - 1), 0, 0)),
            pl.BlockSpec((P * CAP, 128), lambda e: (e, 0)),
        ],
        out_specs=pl.BlockSpec((P * CAP, D), lambda e: (e, 0)),
        out_shape=jax.ShapeDtypeStruct((RD, D), jnp.float32),
    )(disp, We_b, be[:, None, :], wslot)


def _combine(y2d, i0, i1, *, N, D, RD, NC, NW, TOKW, CH):
    mesh = plsc.VectorSubcoreMesh(core_axis_name="c", subcore_axis_name="s")
    NCH = TOKW // CH
    NSEG = D // 16

    @functools.partial(
        pl.kernel,
        mesh=mesh,
        out_type=jax.ShapeDtypeStruct((N, D), jnp.float32),
        scratch_types=[
            pltpu.VMEM((2, CH, D), jnp.float32),
            pltpu.VMEM((2, CH, D), jnp.float32),
            pltpu.VMEM((2, CH, D), jnp.float32),
            pltpu.VMEM((TOKW,), jnp.int32),
            pltpu.VMEM((TOKW,), jnp.int32),
            pltpu.SemaphoreType.DMA,
            pltpu.SemaphoreType.DMA,
            pltpu.SemaphoreType.DMA,
            pltpu.SemaphoreType.DMA,
            pltpu.SemaphoreType.DMA,
            pltpu.SemaphoreType.DMA,
        ],
    )
    def k(y_hbm, i0_hbm, i1_hbm, out_hbm, buf0, buf1, obuf, i0v, i1v,
          sg0a, sg1a, sg0b, sg1b, swa, swb):
        wid = lax.axis_index("s") * NC + lax.axis_index("c")
        base = wid * TOKW
        pltpu.sync_copy(i0_hbm.at[pl.ds(base, TOKW)], i0v)
        pltpu.sync_copy(i1_hbm.at[pl.ds(base, TOKW)], i1v)
        sg = ((sg0a, sg1a), (sg0b, sg1b))
        sw = (swa, swb)

        def issue(c):
            p = c % 2
            sl = pl.ds(c * CH, CH)
            g0 = pltpu.async_copy(y_hbm.at[i0v.at[sl]], buf0.at[p], sg[p][0])
            g1 = pltpu.async_copy(y_hbm.at[i1v.at[sl]], buf1.at[p], sg[p][1])
            return g0, g1

        descs = {0: issue(0)}
        if NCH > 1:
            descs[1] = issue(1)
        writes = {}
        for c in range(NCH):
            p = c % 2
            g0, g1 = descs[c]
            g0.wait()
            g1.wait()
            if c >= 2:
                writes[c - 2].wait()

            def row_body(j, _, p=p):
                for s in range(NSEG):
                    sl = pl.ds(s * 16, 16)
                    obuf[p, j, sl] = buf0[p, j, sl] + buf1[p, j, sl]
                return 0

            lax.fori_loop(0, CH, row_body, 0)
            writes[c] = pltpu.async_copy(
                obuf.at[p], out_hbm.at[pl.ds(base + c * CH, CH)], sw[p])
            if c + 2 < NCH:
                descs[c + 2] = issue(c + 2)
        if NCH > 1:
            writes[NCH - 2].wait()
        writes[NCH - 1].wait()

    return k(y2d, i0, i1)


def kernel(hidden_states, Wg, We, be):
    x = hidden_states
    N, D = x.shape
    E = Wg.shape[1]
    K = 2
    CAP = int(K * N / E * 1.25)
    ZROW = E * CAP
    P = 4
    RD = (E + P) * CAP
    BT = 512
    info = plsc.get_sparse_core_info()
    NC, NS = info.num_cores, info.num_subcores
    NW = NC * NS
    TOKW = N // NW

    i0, i1, w0x, w1x = _router(x, Wg, N=N, D=D, E=E, K=K, CAP=CAP,
                               ZROW=ZROW, BT=BT)
    disp, wslot = _dispatch(x, i0, i1, w0x, w1x, N=N, D=D, RD=RD,
                            NC=NC, NW=NW, TOKW=TOKW)
    y2d = _expert_mm(disp, We, be, wslot, D=D, E=E, CAP=CAP, RD=RD, P=P)
    out = _combine(y2d, i0, i1, N=N, D=D, RD=RD, NC=NC, NW=NW, TOKW=TOKW,
                   CH=16)
    return out


# speedup vs baseline: 1.6661x; 1.0069x over previous
import functools

import jax
import jax.numpy as jnp
from jax import lax
from jax.experimental import pallas as pl
from jax.experimental.pallas import tpu as pltpu
from jax.experimental.pallas import tpu_sc as plsc


def _router(x, Wg, *, N, D, E, K, CAP, ZROW, BT):
    NB = N // BT

    def body(x_ref, wg_ref, i0_ref, i1_ref, w0_ref, w1_ref, hist_ref):
        blk = pl.program_id(0)

        @pl.when(blk == 0)
        def _():
            hist_ref[...] = jnp.zeros_like(hist_ref)

        xb = x_ref[...]
        logitsT = jnp.dot(xb, wg_ref[...],
                          preferred_element_type=jnp.float32).T
        rows = lax.broadcasted_iota(jnp.int32, (E, BT), 0)
        m0 = jnp.max(logitsT, axis=0, keepdims=True)
        idx0 = jnp.min(jnp.where(logitsT == m0, rows, E), axis=0,
                       keepdims=True)
        l2 = jnp.where(rows == idx0, -jnp.inf, logitsT)
        m1 = jnp.max(l2, axis=0, keepdims=True)
        idx1 = jnp.min(jnp.where(l2 == m1, rows, E), axis=0, keepdims=True)
        t = jnp.exp(m1 - m0)
        g0 = 1.0 / (1.0 + t)
        g1 = t / (1.0 + t)

        oh0 = (rows == idx0).astype(jnp.float32)
        oh1 = (rows == idx1).astype(jnp.float32)
        s = oh0 + oh1
        ti = lax.broadcasted_iota(jnp.int32, (BT, BT), 0)
        tj = lax.broadcasted_iota(jnp.int32, (BT, BT), 1)
        triu = (ti < tj).astype(jnp.float32)
        hist = hist_ref[:, 0:1]
        excl = jnp.dot(s, triu, preferred_element_type=jnp.float32) + hist
        pos0 = jnp.sum(excl * oh0, axis=0, keepdims=True)
        pos1 = jnp.sum(excl * oh1, axis=0, keepdims=True)
        hist_ref[:, 0:1] = hist + jnp.sum(s, axis=1, keepdims=True)

        keep0 = pos0 < CAP
        keep1 = pos1 < CAP
        p0 = pos0.astype(jnp.int32)
        p1 = pos1.astype(jnp.int32)
        i0 = jnp.where(keep0, idx0 * CAP + p0, ZROW)
        i1 = jnp.where(keep1, idx1 * CAP + p1, ZROW)
        i0_ref[...] = i0[0]
        i1_ref[...] = i1[0]
        w0_ref[...] = jnp.where(keep0, g0, 0.0)[0]
        w1_ref[...] = jnp.where(keep1, g1, 0.0)[0]

    return pl.pallas_call(
        body,
        grid=(NB,),
        in_specs=[
            pl.BlockSpec((BT, D), lambda i: (i, 0)),
            pl.BlockSpec((D, E), lambda i: (0, 0)),
        ],
        out_specs=[
            pl.BlockSpec((BT,), lambda i: (i,)),
            pl.BlockSpec((BT,), lambda i: (i,)),
            pl.BlockSpec((BT,), lambda i: (i,)),
            pl.BlockSpec((BT,), lambda i: (i,)),
        ],
        out_shape=[
            jax.ShapeDtypeStruct((N,), jnp.int32),
            jax.ShapeDtypeStruct((N,), jnp.int32),
            jax.ShapeDtypeStruct((N,), jnp.float32),
            jax.ShapeDtypeStruct((N,), jnp.float32),
        ],
        scratch_shapes=[pltpu.VMEM((E, 128), jnp.float32)],
    )(x, Wg)


def _dispatch(x, i0, i1, w0, w1, *, N, D, RD, NC, NW, TOKW):
    mesh = plsc.VectorSubcoreMesh(core_axis_name="c", subcore_axis_name="s")

    @functools.partial(
        pl.kernel,
        mesh=mesh,
        out_type=(
            jax.ShapeDtypeStruct((RD, D), jnp.float32),
            jax.ShapeDtypeStruct((RD, 128), jnp.float32),
        ),
        scratch_types=[
            pltpu.VMEM((TOKW, D), jnp.float32),
            pltpu.VMEM((TOKW, 128), jnp.float32),
            pltpu.VMEM((TOKW,), jnp.int32),
            pltpu.VMEM((TOKW,), jnp.int32),
            pltpu.SemaphoreType.DMA,
            pltpu.SemaphoreType.DMA,
        ],
    )
    def k(x_hbm, i0_hbm, i1_hbm, w0x_hbm, w1x_hbm, disp_hbm, wslot_hbm,
          xbuf, wbuf, i0v, i1v, sem, wsem):
        wid = lax.axis_index("s") * NC + lax.axis_index("c")
        base = wid * TOKW
        sl = pl.ds(base, TOKW)
        pltpu.sync_copy(i0_hbm.at[sl], i0v)
        pltpu.sync_copy(i1_hbm.at[sl], i1v)
        pltpu.sync_copy(x_hbm.at[sl], xbuf)
        pltpu.sync_copy(w0x_hbm.at[sl], wbuf)
        c0 = pltpu.async_copy(xbuf, disp_hbm.at[i0v], sem)
        c2 = pltpu.async_copy(xbuf, disp_hbm.at[i1v], sem)
        c1 = pltpu.async_copy(wbuf, wslot_hbm.at[i0v], wsem)
        c1.wait()
        pltpu.sync_copy(w1x_hbm.at[sl], wbuf)
        c3 = pltpu.async_copy(wbuf, wslot_hbm.at[i1v], wsem)
        c0.wait()
        c2.wait()
        c3.wait()

    return k(x, i0, i1,
             jnp.broadcast_to(w0[:, None], (N, 128)),
             jnp.broadcast_to(w1[:, None], (N, 128)))


def _expert_mm(disp, We_b, be, wslot, *, D, E, CAP, RD, P):
    G = (E + P) // P

    def body(disp_ref, we_ref, be_ref, ws_ref, y_ref):
        for p in range(P):
            rows = pl.ds(p * CAP, CAP)
            xb = disp_ref[rows, :].astype(We_b.dtype)
            y = jnp.dot(xb, we_ref[p], preferred_element_type=jnp.float32)
            y_ref[rows, :] = (y + be_ref[p]) * ws_ref[rows, 0:1]

    return pl.pallas_call(
        body,
        grid=(G,),
        in_specs=[
            pl.BlockSpec((P * CAP, D), lambda e: (e, 0)),
            pl.BlockSpec((P, D, D), lambda e: (jnp.minimum(e, E // P - 1), 0, 0)),
            pl.BlockSpec((P, 1, D), lambda e: (jnp.minimum(e, E // P - 1), 0, 0)),
            pl.BlockSpec((P * CAP, 128), lambda e: (e, 0)),
        ],
        out_specs=pl.BlockSpec((P * CAP, D), lambda e: (e, 0)),
        out_shape=jax.ShapeDtypeStruct((RD, D), jnp.float32),
    )(disp, We_b, be[:, None, :], wslot)


def _combine(y2d, i0, i1, *, N, D, RD, NC, NW, TOKW, CH):
    mesh = plsc.VectorSubcoreMesh(core_axis_name="c", subcore_axis_name="s")
    NCH = TOKW // CH
    NSEG = D // 16

    @functools.partial(
        pl.kernel,
        mesh=mesh,
        out_type=jax.ShapeDtypeStruct((N, D), jnp.float32),
        scratch_types=[
            pltpu.VMEM((2, CH, D), jnp.float32),
            pltpu.VMEM((2, CH, D), jnp.float32),
            pltpu.VMEM((2, CH, D), jnp.float32),
            pltpu.VMEM((TOKW,), jnp.int32),
            pltpu.VMEM((TOKW,), jnp.int32),
            pltpu.SemaphoreType.DMA,
            pltpu.SemaphoreType.DMA,
            pltpu.SemaphoreType.DMA,
            pltpu.SemaphoreType.DMA,
            pltpu.SemaphoreType.DMA,
            pltpu.SemaphoreType.DMA,
        ],
    )
    def k(y_hbm, i0_hbm, i1_hbm, out_hbm, buf0, buf1, obuf, i0v, i1v,
          sg0a, sg1a, sg0b, sg1b, swa, swb):
        wid = lax.axis_index("s") * NC + lax.axis_index("c")
        base = wid * TOKW
        pltpu.sync_copy(i0_hbm.at[pl.ds(base, TOKW)], i0v)
        pltpu.sync_copy(i1_hbm.at[pl.ds(base, TOKW)], i1v)
        sg = ((sg0a, sg1a), (sg0b, sg1b))
        sw = (swa, swb)

        def issue(c):
            p = c % 2
            sl = pl.ds(c * CH, CH)
            g0 = pltpu.async_copy(y_hbm.at[i0v.at[sl]], buf0.at[p], sg[p][0])
            g1 = pltpu.async_copy(y_hbm.at[i1v.at[sl]], buf1.at[p], sg[p][1])
            return g0, g1

        descs = {0: issue(0)}
        if NCH > 1:
            descs[1] = issue(1)
        writes = {}
        for c in range(NCH):
            p = c % 2
            g0, g1 = descs[c]
            g0.wait()
            g1.wait()
            if c >= 2:
                writes[c - 2].wait()

            def row_body(j, _, p=p):
                for s in range(NSEG):
                    sl = pl.ds(s * 16, 16)
                    obuf[p, j, sl] = buf0[p, j, sl] + buf1[p, j, sl]
                return 0

            lax.fori_loop(0, CH, row_body, 0)
            writes[c] = pltpu.async_copy(
                obuf.at[p], out_hbm.at[pl.ds(base + c * CH, CH)], sw[p])
            if c + 2 < NCH:
                descs[c + 2] = issue(c + 2)
        if NCH > 1:
            writes[NCH - 2].wait()
        writes[NCH - 1].wait()

    return k(y2d, i0, i1)


def kernel(hidden_states, Wg, We, be):
    x = hidden_states
    N, D = x.shape
    E = Wg.shape[1]
    K = 2
    CAP = int(K * N / E * 1.25)
    ZROW = E * CAP
    P = 4
    RD = (E + P) * CAP
    BT = 512
    info = plsc.get_sparse_core_info()
    NC, NS = info.num_cores, info.num_subcores
    NW = NC * NS
    TOKW = N // NW

    i0, i1, w0x, w1x = _router(x, Wg, N=N, D=D, E=E, K=K, CAP=CAP,
                               ZROW=ZROW, BT=BT)
    disp, wslot = _dispatch(x, i0, i1, w0x, w1x, N=N, D=D, RD=RD,
                            NC=NC, NW=NW, TOKW=TOKW)
    y2d = _expert_mm(disp, We, be, wslot, D=D, E=E, CAP=CAP, RD=RD, P=P)
    out = _combine(y2d, i0, i1, N=N, D=D, RD=RD, NC=NC, NW=NW, TOKW=TOKW,
                   CH=16)
    return out


# speedup vs baseline: 1.6816x; 1.0093x over previous
import functools

import jax
import jax.numpy as jnp
from jax import lax
from jax.experimental import pallas as pl
from jax.experimental.pallas import tpu as pltpu
from jax.experimental.pallas import tpu_sc as plsc


def _router(x, Wg, *, N, D, E, K, CAP, ZROW, BT):
    NB = N // BT

    def body(x_ref, wg_ref, i0_ref, i1_ref, w0_ref, w1_ref, hist_ref):
        blk = pl.program_id(0)

        @pl.when(blk == 0)
        def _():
            hist_ref[...] = jnp.zeros_like(hist_ref)

        xb = x_ref[...]
        logitsT = jnp.dot(xb, wg_ref[...],
                          preferred_element_type=jnp.float32).T
        rows = lax.broadcasted_iota(jnp.int32, (E, BT), 0)
        m0 = jnp.max(logitsT, axis=0, keepdims=True)
        idx0 = jnp.min(jnp.where(logitsT == m0, rows, E), axis=0,
                       keepdims=True)
        l2 = jnp.where(rows == idx0, -jnp.inf, logitsT)
        m1 = jnp.max(l2, axis=0, keepdims=True)
        idx1 = jnp.min(jnp.where(l2 == m1, rows, E), axis=0, keepdims=True)
        t = jnp.exp(m1 - m0)
        g0 = 1.0 / (1.0 + t)
        g1 = t / (1.0 + t)

        oh0 = (rows == idx0).astype(jnp.float32)
        oh1 = (rows == idx1).astype(jnp.float32)
        s = oh0 + oh1
        ti = lax.broadcasted_iota(jnp.int32, (BT, BT), 0)
        tj = lax.broadcasted_iota(jnp.int32, (BT, BT), 1)
        triu = (ti < tj).astype(jnp.float32)
        hist = hist_ref[:, 0:1]
        excl = jnp.dot(s, triu, preferred_element_type=jnp.float32) + hist
        pos0 = jnp.sum(excl * oh0, axis=0, keepdims=True)
        pos1 = jnp.sum(excl * oh1, axis=0, keepdims=True)
        hist_ref[:, 0:1] = hist + jnp.sum(s, axis=1, keepdims=True)

        keep0 = pos0 < CAP
        keep1 = pos1 < CAP
        p0 = pos0.astype(jnp.int32)
        p1 = pos1.astype(jnp.int32)
        i0 = jnp.where(keep0, idx0 * CAP + p0, ZROW)
        i1 = jnp.where(keep1, idx1 * CAP + p1, ZROW)
        i0_ref[...] = i0[0]
        i1_ref[...] = i1[0]
        w0 = jnp.where(keep0, g0, 0.0)
        w1 = jnp.where(keep1, g1, 0.0)
        w0_ref[...] = jnp.broadcast_to(w0.T, (BT, 128))
        w1_ref[...] = jnp.broadcast_to(w1.T, (BT, 128))

    return pl.pallas_call(
        body,
        grid=(NB,),
        in_specs=[
            pl.BlockSpec((BT, D), lambda i: (i, 0)),
            pl.BlockSpec((D, E), lambda i: (0, 0)),
        ],
        out_specs=[
            pl.BlockSpec((BT,), lambda i: (i,)),
            pl.BlockSpec((BT,), lambda i: (i,)),
            pl.BlockSpec((BT, 128), lambda i: (i, 0)),
            pl.BlockSpec((BT, 128), lambda i: (i, 0)),
        ],
        out_shape=[
            jax.ShapeDtypeStruct((N,), jnp.int32),
            jax.ShapeDtypeStruct((N,), jnp.int32),
            jax.ShapeDtypeStruct((N, 128), jnp.float32),
            jax.ShapeDtypeStruct((N, 128), jnp.float32),
        ],
        scratch_shapes=[pltpu.VMEM((E, 128), jnp.float32)],
    )(x, Wg)


def _dispatch(x, i0, i1, w0, w1, *, N, D, RD, NC, NW, TOKW):
    mesh = plsc.VectorSubcoreMesh(core_axis_name="c", subcore_axis_name="s")

    @functools.partial(
        pl.kernel,
        mesh=mesh,
        out_type=(
            jax.ShapeDtypeStruct((RD, D), jnp.float32),
            jax.ShapeDtypeStruct((RD, 128), jnp.float32),
        ),
        scratch_types=[
            pltpu.VMEM((TOKW, D), jnp.float32),
            pltpu.VMEM((TOKW, 128), jnp.float32),
            pltpu.VMEM((TOKW,), jnp.int32),
            pltpu.VMEM((TOKW,), jnp.int32),
            pltpu.SemaphoreType.DMA,
            pltpu.SemaphoreType.DMA,
        ],
    )
    def k(x_hbm, i0_hbm, i1_hbm, w0x_hbm, w1x_hbm, disp_hbm, wslot_hbm,
          xbuf, wbuf, i0v, i1v, sem, wsem):
        wid = lax.axis_index("s") * NC + lax.axis_index("c")
        base = wid * TOKW
        sl = pl.ds(base, TOKW)
        pltpu.sync_copy(i0_hbm.at[sl], i0v)
        pltpu.sync_copy(i1_hbm.at[sl], i1v)
        pltpu.sync_copy(x_hbm.at[sl], xbuf)
        pltpu.sync_copy(w0x_hbm.at[sl], wbuf)
        c0 = pltpu.async_copy(xbuf, disp_hbm.at[i0v], sem)
        c2 = pltpu.async_copy(xbuf, disp_hbm.at[i1v], sem)
        c1 = pltpu.async_copy(wbuf, wslot_hbm.at[i0v], wsem)
        c1.wait()
        pltpu.sync_copy(w1x_hbm.at[sl], wbuf)
        c3 = pltpu.async_copy(wbuf, wslot_hbm.at[i1v], wsem)
        c0.wait()
        c2.wait()
        c3.wait()

    return k(x, i0, i1, w0, w1)


def _expert_mm(disp, We_b, be, wslot, *, D, E, CAP, RD, P):
    G = (E + P) // P

    def body(disp_ref, we_ref, be_ref, ws_ref, y_ref):
        for p in range(P):
            rows = pl.ds(p * CAP, CAP)
            xb = disp_ref[rows, :].astype(We_b.dtype)
            y = jnp.dot(xb, we_ref[p], preferred_element_type=jnp.float32)
            y_ref[rows, :] = (y + be_ref[p]) * ws_ref[rows, 0:1]

    return pl.pallas_call(
        body,
        grid=(G,),
        in_specs=[
            pl.BlockSpec((P * CAP, D), lambda e: (e, 0)),
            pl.BlockSpec((P, D, D), lambda e: (jnp.minimum(e, E // P - 1), 0, 0)),
            pl.BlockSpec((P, 1, D), lambda e: (jnp.minimum(e, E // P - 1), 0, 0)),
            pl.BlockSpec((P * CAP, 128), lambda e: (e, 0)),
        ],
        out_specs=pl.BlockSpec((P * CAP, D), lambda e: (e, 0)),
        out_shape=jax.ShapeDtypeStruct((RD, D), jnp.float32),
    )(disp, We_b, be[:, None, :], wslot)


def _combine(y2d, i0, i1, *, N, D, RD, NC, NW, TOKW, CH):
    mesh = plsc.VectorSubcoreMesh(core_axis_name="c", subcore_axis_name="s")
    NCH = TOKW // CH
    NSEG = D // 16

    @functools.partial(
        pl.kernel,
        mesh=mesh,
        out_type=jax.ShapeDtypeStruct((N, D), jnp.float32),
        scratch_types=[
            pltpu.VMEM((2, CH, D), jnp.float32),
            pltpu.VMEM((2, CH, D), jnp.float32),
            pltpu.VMEM((2, CH, D), jnp.float32),
            pltpu.VMEM((TOKW,), jnp.int32),
            pltpu.VMEM((TOKW,), jnp.int32),
            pltpu.SemaphoreType.DMA,
            pltpu.SemaphoreType.DMA,
            pltpu.SemaphoreType.DMA,
            pltpu.SemaphoreType.DMA,
            pltpu.SemaphoreType.DMA,
            pltpu.SemaphoreType.DMA,
        ],
    )
    def k(y_hbm, i0_hbm, i1_hbm, out_hbm, buf0, buf1, obuf, i0v, i1v,
          sg0a, sg1a, sg0b, sg1b, swa, swb):
        wid = lax.axis_index("s") * NC + lax.axis_index("c")
        base = wid * TOKW
        pltpu.sync_copy(i0_hbm.at[pl.ds(base, TOKW)], i0v)
        pltpu.sync_copy(i1_hbm.at[pl.ds(base, TOKW)], i1v)
        sg = ((sg0a, sg1a), (sg0b, sg1b))
        sw = (swa, swb)

        def issue(c):
            p = c % 2
            sl = pl.ds(c * CH, CH)
            g0 = pltpu.async_copy(y_hbm.at[i0v.at[sl]], buf0.at[p], sg[p][0])
            g1 = pltpu.async_copy(y_hbm.at[i1v.at[sl]], buf1.at[p], sg[p][1])
            return g0, g1

        descs = {0: issue(0)}
        if NCH > 1:
            descs[1] = issue(1)
        writes = {}
        for c in range(NCH):
            p = c % 2
            g0, g1 = descs[c]
            g0.wait()
            g1.wait()
            if c >= 2:
                writes[c - 2].wait()

            def row_body(j, _, p=p):
                for s in range(NSEG):
                    sl = pl.ds(s * 16, 16)
                    obuf[p, j, sl] = buf0[p, j, sl] + buf1[p, j, sl]
                return 0

            lax.fori_loop(0, CH, row_body, 0)
            writes[c] = pltpu.async_copy(
                obuf.at[p], out_hbm.at[pl.ds(base + c * CH, CH)], sw[p])
            if c + 2 < NCH:
                descs[c + 2] = issue(c + 2)
        if NCH > 1:
            writes[NCH - 2].wait()
        writes[NCH - 1].wait()

    return k(y2d, i0, i1)


def kernel(hidden_states, Wg, We, be):
    x = hidden_states
    N, D = x.shape
    E = Wg.shape[1]
    K = 2
    CAP = int(K * N / E * 1.25)
    ZROW = E * CAP
    P = 8
    RD = (E + P) * CAP
    BT = 512
    info = plsc.get_sparse_core_info()
    NC, NS = info.num_cores, info.num_subcores
    NW = NC * NS
    TOKW = N // NW

    i0, i1, w0x, w1x = _router(x, Wg, N=N, D=D, E=E, K=K, CAP=CAP,
                               ZROW=ZROW, BT=BT)
    disp, wslot = _dispatch(x, i0, i1, w0x, w1x, N=N, D=D, RD=RD,
                            NC=NC, NW=NW, TOKW=TOKW)
    y2d = _expert_mm(disp, We, be, wslot, D=D, E=E, CAP=CAP, RD=RD, P=P)
    out = _combine(y2d, i0, i1, N=N, D=D, RD=RD, NC=NC, NW=NW, TOKW=TOKW,
                   CH=16)
    return out


# speedup vs baseline: 1.6995x; 1.0106x over previous
import functools

import jax
import jax.numpy as jnp
from jax import lax
from jax.experimental import pallas as pl
from jax.experimental.pallas import tpu as pltpu
from jax.experimental.pallas import tpu_sc as plsc


def _router(x, Wg, *, N, D, E, K, CAP, ZROW, BT):
    NB = N // BT

    def body(x_ref, wg_ref, i0_ref, i1_ref, w0_ref, w1_ref, hist_ref):
        blk = pl.program_id(0)

        @pl.when(blk == 0)
        def _():
            hist_ref[...] = jnp.zeros_like(hist_ref)

        xb = x_ref[...]
        logitsT = jnp.dot(xb, wg_ref[...],
                          preferred_element_type=jnp.float32).T
        rows = lax.broadcasted_iota(jnp.int32, (E, BT), 0)
        m0 = jnp.max(logitsT, axis=0, keepdims=True)
        idx0 = jnp.min(jnp.where(logitsT == m0, rows, E), axis=0,
                       keepdims=True)
        l2 = jnp.where(rows == idx0, -jnp.inf, logitsT)
        m1 = jnp.max(l2, axis=0, keepdims=True)
        idx1 = jnp.min(jnp.where(l2 == m1, rows, E), axis=0, keepdims=True)
        t = jnp.exp(m1 - m0)
        g0 = 1.0 / (1.0 + t)
        g1 = t / (1.0 + t)

        oh0 = (rows == idx0).astype(jnp.float32)
        oh1 = (rows == idx1).astype(jnp.float32)
        s = oh0 + oh1
        ti = lax.broadcasted_iota(jnp.int32, (BT, BT), 0)
        tj = lax.broadcasted_iota(jnp.int32, (BT, BT), 1)
        triu = (ti < tj).astype(jnp.float32)
        hist = hist_ref[:, 0:1]
        excl = jnp.dot(s, triu, preferred_element_type=jnp.float32) + hist
        pos0 = jnp.sum(excl * oh0, axis=0, keepdims=True)
        pos1 = jnp.sum(excl * oh1, axis=0, keepdims=True)
        hist_ref[:, 0:1] = hist + jnp.sum(s, axis=1, keepdims=True)

        keep0 = pos0 < CAP
        keep1 = pos1 < CAP
        p0 = pos0.astype(jnp.int32)
        p1 = pos1.astype(jnp.int32)
        i0 = jnp.where(keep0, idx0 * CAP + p0, ZROW)
        i1 = jnp.where(keep1, idx1 * CAP + p1, ZROW)
        i0_ref[...] = i0[0]
        i1_ref[...] = i1[0]
        w0 = jnp.where(keep0, g0, 0.0)
        w1 = jnp.where(keep1, g1, 0.0)
        w0_ref[...] = jnp.broadcast_to(w0.T, (BT, 128))
        w1_ref[...] = jnp.broadcast_to(w1.T, (BT, 128))

    return pl.pallas_call(
        body,
        grid=(NB,),
        in_specs=[
            pl.BlockSpec((BT, D), lambda i: (i, 0)),
            pl.BlockSpec((D, E), lambda i: (0, 0)),
        ],
        out_specs=[
            pl.BlockSpec((BT,), lambda i: (i,)),
            pl.BlockSpec((BT,), lambda i: (i,)),
            pl.BlockSpec((BT, 128), lambda i: (i, 0)),
            pl.BlockSpec((BT, 128), lambda i: (i, 0)),
        ],
        out_shape=[
            jax.ShapeDtypeStruct((N,), jnp.int32),
            jax.ShapeDtypeStruct((N,), jnp.int32),
            jax.ShapeDtypeStruct((N, 128), jnp.float32),
            jax.ShapeDtypeStruct((N, 128), jnp.float32),
        ],
        scratch_shapes=[pltpu.VMEM((E, 128), jnp.float32)],
    )(x, Wg)


def _dispatch(x, i0, i1, w0, w1, *, N, D, RD, NC, NW, TOKW):
    mesh = plsc.VectorSubcoreMesh(core_axis_name="c", subcore_axis_name="s")

    @functools.partial(
        pl.kernel,
        mesh=mesh,
        out_type=(
            jax.ShapeDtypeStruct((RD, D), jnp.float32),
            jax.ShapeDtypeStruct((RD, 128), jnp.float32),
        ),
        scratch_types=[
            pltpu.VMEM((TOKW, D), jnp.float32),
            pltpu.VMEM((TOKW, 128), jnp.float32),
            pltpu.VMEM((TOKW,), jnp.int32),
            pltpu.VMEM((TOKW,), jnp.int32),
            pltpu.SemaphoreType.DMA,
            pltpu.SemaphoreType.DMA,
        ],
    )
    def k(x_hbm, i0_hbm, i1_hbm, w0x_hbm, w1x_hbm, disp_hbm, wslot_hbm,
          xbuf, wbuf, i0v, i1v, sem, wsem):
        wid = lax.axis_index("s") * NC + lax.axis_index("c")
        base = wid * TOKW
        sl = pl.ds(base, TOKW)
        pltpu.sync_copy(i0_hbm.at[sl], i0v)
        pltpu.sync_copy(i1_hbm.at[sl], i1v)
        pltpu.sync_copy(x_hbm.at[sl], xbuf)
        pltpu.sync_copy(w0x_hbm.at[sl], wbuf)
        c0 = pltpu.async_copy(xbuf, disp_hbm.at[i0v], sem)
        c2 = pltpu.async_copy(xbuf, disp_hbm.at[i1v], sem)
        c1 = pltpu.async_copy(wbuf, wslot_hbm.at[i0v], wsem)
        c1.wait()
        pltpu.sync_copy(w1x_hbm.at[sl], wbuf)
        c3 = pltpu.async_copy(wbuf, wslot_hbm.at[i1v], wsem)
        c0.wait()
        c2.wait()
        c3.wait()

    return k(x, i0, i1, w0, w1)


def _expert_mm(disp, We_b, be, wslot, *, D, E, CAP, RD, P):
    G = (E + P) // P

    def body(disp_ref, we_ref, be_ref, ws_ref, y_ref):
        for p in range(P):
            rows = pl.ds(p * CAP, CAP)
            xb = disp_ref[rows, :].astype(We_b.dtype)
            y = jnp.dot(xb, we_ref[p], preferred_element_type=jnp.float32)
            y_ref[rows, :] = (y + be_ref[p]) * ws_ref[rows, 0:1]

    return pl.pallas_call(
        body,
        grid=(G,),
        in_specs=[
            pl.BlockSpec((P * CAP, D), lambda e: (e, 0)),
            pl.BlockSpec((P, D, D), lambda e: (jnp.minimum(e, E // P - 1), 0, 0)),
            pl.BlockSpec((P, 1, D), lambda e: (jnp.minimum(e, E // P - 1), 0, 0)),
            pl.BlockSpec((P * CAP, 128), lambda e: (e, 0)),
        ],
        out_specs=pl.BlockSpec((P * CAP, D), lambda e: (e, 0)),
        out_shape=jax.ShapeDtypeStruct((RD, D), jnp.float32),
    )(disp, We_b, be[:, None, :], wslot)


def _combine(y2d, i0, i1, *, N, D, RD, NC, NW, TOKW, CH):
    mesh = plsc.VectorSubcoreMesh(core_axis_name="c", subcore_axis_name="s")
    NCH = TOKW // CH
    NSEG = D // 16

    @functools.partial(
        pl.kernel,
        mesh=mesh,
        out_type=jax.ShapeDtypeStruct((N, D), jnp.float32),
        scratch_types=[
            pltpu.VMEM((2, CH, D), jnp.float32),
            pltpu.VMEM((2, CH, D), jnp.float32),
            pltpu.VMEM((2, CH, D), jnp.float32),
            pltpu.VMEM((TOKW,), jnp.int32),
            pltpu.VMEM((TOKW,), jnp.int32),
            pltpu.SemaphoreType.DMA,
            pltpu.SemaphoreType.DMA,
            pltpu.SemaphoreType.DMA,
            pltpu.SemaphoreType.DMA,
            pltpu.SemaphoreType.DMA,
            pltpu.SemaphoreType.DMA,
        ],
    )
    def k(y_hbm, i0_hbm, i1_hbm, out_hbm, buf0, buf1, obuf, i0v, i1v,
          sg0a, sg1a, sg0b, sg1b, swa, swb):
        wid = lax.axis_index("s") * NC + lax.axis_index("c")
        base = wid * TOKW
        pltpu.sync_copy(i0_hbm.at[pl.ds(base, TOKW)], i0v)
        pltpu.sync_copy(i1_hbm.at[pl.ds(base, TOKW)], i1v)
        sg = ((sg0a, sg1a), (sg0b, sg1b))
        sw = (swa, swb)

        def issue(c):
            p = c % 2
            sl = pl.ds(c * CH, CH)
            g0 = pltpu.async_copy(y_hbm.at[i0v.at[sl]], buf0.at[p], sg[p][0])
            g1 = pltpu.async_copy(y_hbm.at[i1v.at[sl]], buf1.at[p], sg[p][1])
            return g0, g1

        descs = {0: issue(0)}
        if NCH > 1:
            descs[1] = issue(1)
        writes = {}
        for c in range(NCH):
            p = c % 2
            g0, g1 = descs[c]
            g0.wait()
            g1.wait()
            if c >= 2:
                writes[c - 2].wait()

            def row_body(j, _, p=p):
                for s in range(NSEG):
                    sl = pl.ds(s * 16, 16)
                    obuf[p, j, sl] = buf0[p, j, sl] + buf1[p, j, sl]
                return 0

            lax.fori_loop(0, CH, row_body, 0)
            writes[c] = pltpu.async_copy(
                obuf.at[p], out_hbm.at[pl.ds(base + c * CH, CH)], sw[p])
            if c + 2 < NCH:
                descs[c + 2] = issue(c + 2)
        if NCH > 1:
            writes[NCH - 2].wait()
        writes[NCH - 1].wait()

    return k(y2d, i0, i1)


def kernel(hidden_states, Wg, We, be):
    x = hidden_states
    N, D = x.shape
    E = Wg.shape[1]
    K = 2
    CAP = int(K * N / E * 1.25)
    ZROW = E * CAP
    P = 4
    RD = (E + P) * CAP
    BT = 512
    info = plsc.get_sparse_core_info()
    NC, NS = info.num_cores, info.num_subcores
    NW = NC * NS
    TOKW = N // NW

    i0, i1, w0x, w1x = _router(x, Wg, N=N, D=D, E=E, K=K, CAP=CAP,
                               ZROW=ZROW, BT=BT)
    disp, wslot = _dispatch(x, i0, i1, w0x, w1x, N=N, D=D, RD=RD,
                            NC=NC, NW=NW, TOKW=TOKW)
    y2d = _expert_mm(disp, We, be, wslot, D=D, E=E, CAP=CAP, RD=RD, P=P)
    out = _combine(y2d, i0, i1, N=N, D=D, RD=RD, NC=NC, NW=NW, TOKW=TOKW,
                   CH=16)
    return out
